# initial kernel scaffold (unmeasured)
import jax
import jax.numpy as jnp
from jax import lax
from jax.experimental import pallas as pl
from jax.experimental.pallas import tpu as pltpu

N_DEV = 16


def kernel(x, w_mat, scale_x, scale_w):
    m, _ = x.shape
    _, n = w_mat.shape
    ch = m // N_DEV

    def body(x_ref, w_ref, sx_ref, sw_ref, out_ref,
             comm_ref, send_sems, recv_sems, credit_sems):
        me = lax.axis_index("i")
        left = lax.rem(me + N_DEV - 1, N_DEV)
        right = lax.rem(me + 1, N_DEV)

        s = sx_ref[0] * sw_ref[0]
        xb = x_ref[...].astype(jnp.bfloat16)
        wb = w_ref[...].astype(jnp.bfloat16)
        out_ref[...] = jnp.dot(xb, wb, preferred_element_type=jnp.float32) * s

        def out_chunk(idx):
            return out_ref.at[pl.ds(idx * ch, ch), :]

        n_steps = 2 * (N_DEV - 1)
        for step in range(n_steps):
            slot = step % 2
            rs = step < N_DEV - 1
            if rs:
                send_idx = lax.rem(me - step + N_DEV, N_DEV)
                recv_idx = lax.rem(me - step - 1 + N_DEV, N_DEV)
                dst = comm_ref.at[slot]
            else:
                t = step - (N_DEV - 1)
                send_idx = lax.rem(me + 1 - t + N_DEV, N_DEV)
                dst = out_chunk(send_idx)

            if step >= 2:
                pl.semaphore_wait(credit_sems.at[slot], 1)

            rdma = pltpu.make_async_remote_copy(
                src_ref=out_chunk(send_idx),
                dst_ref=dst,
                send_sem=send_sems.at[slot],
                recv_sem=recv_sems.at[slot],
                device_id=(right,),
                device_id_type=pl.DeviceIdType.MESH,
            )
            rdma.start()
            rdma.wait()

            if rs:
                out_ref[pl.ds(recv_idx * ch, ch), :] = (
                    out_ref[pl.ds(recv_idx * ch, ch), :] + comm_ref[slot]
                )
            if step < n_steps - 2:
                pl.semaphore_signal(
                    credit_sems.at[slot], inc=1,
                    device_id=(left,), device_id_type=pl.DeviceIdType.MESH,
                )

    return pl.pallas_call(
        body,
        out_shape=jax.ShapeDtypeStruct((m, n), jnp.float32),
        in_specs=[
            pl.BlockSpec(memory_space=pltpu.VMEM),
            pl.BlockSpec(memory_space=pltpu.VMEM),
            pl.BlockSpec(memory_space=pltpu.SMEM),
            pl.BlockSpec(memory_space=pltpu.SMEM),
        ],
        out_specs=pl.BlockSpec(memory_space=pltpu.VMEM),
        scratch_shapes=[
            pltpu.VMEM((2, ch, n), jnp.float32),
            pltpu.SemaphoreType.DMA((2,)),
            pltpu.SemaphoreType.DMA((2,)),
            pltpu.SemaphoreType.REGULAR((2,)),
        ],
        compiler_params=pltpu.CompilerParams(collective_id=0),
    )(x, w_mat, scale_x, scale_w)


# baseline (device time: 784027 ns/iter reference)
import jax
import jax.numpy as jnp
from jax import lax
from jax.experimental import pallas as pl
from jax.experimental.pallas import tpu as pltpu

N_DEV = 16


def kernel(x, w_mat, scale_x, scale_w):
    m, _ = x.shape
    _, n = w_mat.shape
    ch = m // N_DEV

    def body(x_ref, w_ref, sx_ref, sw_ref, out_ref,
             comm_ref, send_sems, recv_sems, credit_sems):
        me = lax.axis_index("i")
        left = lax.rem(me + N_DEV - 1, N_DEV)
        right = lax.rem(me + 1, N_DEV)

        s = sx_ref[0] * sw_ref[0]
        xb = x_ref[...].astype(jnp.bfloat16)
        wb = w_ref[...].astype(jnp.bfloat16)
        out_ref[...] = jnp.dot(xb, wb, preferred_element_type=jnp.float32) * s

        def out_chunk(idx):
            return out_ref.at[pl.ds(idx * ch, ch), :]

        n_steps = 2 * (N_DEV - 1)
        for step in range(n_steps):
            slot = step % 2
            rs = step < N_DEV - 1
            if rs:
                send_idx = lax.rem(me - step + N_DEV, N_DEV)
                recv_idx = lax.rem(me - step - 1 + N_DEV, N_DEV)
                dst = comm_ref.at[slot]
            else:
                t = step - (N_DEV - 1)
                send_idx = lax.rem(me + 1 - t + N_DEV, N_DEV)
                dst = out_chunk(send_idx)

            if step >= 2:
                pl.semaphore_wait(credit_sems.at[slot], 1)

            rdma = pltpu.make_async_remote_copy(
                src_ref=out_chunk(send_idx),
                dst_ref=dst,
                send_sem=send_sems.at[slot],
                recv_sem=recv_sems.at[slot],
                device_id=(right,),
                device_id_type=pl.DeviceIdType.MESH,
            )
            rdma.start()
            rdma.wait()

            if rs:
                out_ref[pl.ds(recv_idx * ch, ch), :] = (
                    out_ref[pl.ds(recv_idx * ch, ch), :] + comm_ref[slot]
                )
            if step < n_steps - 2:
                pl.semaphore_signal(
                    credit_sems.at[slot], inc=1,
                    device_id=(left,), device_id_type=pl.DeviceIdType.MESH,
                )

    return pl.pallas_call(
        body,
        out_shape=jax.ShapeDtypeStruct((m, n), jnp.float32),
        in_specs=[
            pl.BlockSpec(memory_space=pltpu.VMEM),
            pl.BlockSpec(memory_space=pltpu.VMEM),
            pl.BlockSpec(memory_space=pltpu.SMEM),
            pl.BlockSpec(memory_space=pltpu.SMEM),
        ],
        out_specs=pl.BlockSpec(memory_space=pltpu.VMEM),
        scratch_shapes=[
            pltpu.VMEM((2, ch, n), jnp.float32),
            pltpu.SemaphoreType.DMA((2,)),
            pltpu.SemaphoreType.DMA((2,)),
            pltpu.SemaphoreType.REGULAR((2,)),
        ],
        compiler_params=pltpu.CompilerParams(
            vmem_limit_bytes=100 * 1024 * 1024,
        ),
    )(x, w_mat, scale_x, scale_w)


# device time: 442950 ns/iter; 1.7700x vs baseline; 1.7700x over previous
import jax
import jax.numpy as jnp
from jax import lax
from jax.experimental import pallas as pl
from jax.experimental.pallas import tpu as pltpu

N_DEV = 16

_MESH_COORDS = [(x, y, z) for z in range(4) for (x, y) in ((0, 0), (1, 0), (1, 1), (0, 1))]
_HAM = [(0, 0, 0), (0, 1, 0), (0, 1, 1), (0, 0, 1),
        (0, 0, 2), (0, 1, 2), (0, 1, 3), (0, 0, 3),
        (1, 0, 3), (1, 1, 3), (1, 1, 2), (1, 0, 2),
        (1, 0, 1), (1, 1, 1), (1, 1, 0), (1, 0, 0)]
_PI = [_MESH_COORDS.index(c) for c in _HAM]
_RING = [0] * N_DEV
_SUCC = [0] * N_DEV
_PRED = [0] * N_DEV
for _r, _m in enumerate(_PI):
    _RING[_m] = _r
    _SUCC[_m] = _PI[(_r + 1) % N_DEV]
    _PRED[_m] = _PI[(_r - 1) % N_DEV]


def kernel(x, w_mat, scale_x, scale_w):
    m, _ = x.shape
    _, n = w_mat.shape
    half = m // 2
    ch = half // N_DEV

    ring_tab = jnp.array(_RING, dtype=jnp.int32)
    succ_tab = jnp.array(_SUCC, dtype=jnp.int32)
    pred_tab = jnp.array(_PRED, dtype=jnp.int32)

    def body(x_ref, w_ref, sx_ref, sw_ref, ring_ref, succ_ref, pred_ref,
             out_ref, comm_f, comm_b,
             send_f, recv_f, cred_f, send_b, recv_b, cred_b):
        me = lax.axis_index("i")
        r = ring_ref[me]
        nxt = succ_ref[me]
        prv = pred_ref[me]

        s = sx_ref[0] * sw_ref[0]
        xb = x_ref[...].astype(jnp.bfloat16)
        wb = w_ref[...].astype(jnp.bfloat16)
        out_ref[...] = jnp.dot(xb, wb, preferred_element_type=jnp.float32) * s

        def chunk_f(idx):
            return out_ref.at[pl.ds(idx * ch, ch), :]

        def chunk_b(idx):
            return out_ref.at[pl.ds(half + idx * ch, ch), :]

        n_steps = 2 * (N_DEV - 1)
        for step in range(n_steps):
            slot = step % 2
            rs = step < N_DEV - 1
            if rs:
                sa = lax.rem(r - step + N_DEV, N_DEV)
                ra = lax.rem(r - step - 1 + N_DEV, N_DEV)
                sb = lax.rem(r + step, N_DEV)
                rb = lax.rem(r + step + 1, N_DEV)
                dst_f = comm_f.at[slot]
                dst_b = comm_b.at[slot]
            else:
                t = step - (N_DEV - 1)
                sa = lax.rem(r + 1 - t + N_DEV, N_DEV)
                sb = lax.rem(r - 1 + t + N_DEV, N_DEV)
                dst_f = chunk_f(sa)
                dst_b = chunk_b(sb)

            if step >= 2:
                pl.semaphore_wait(cred_f.at[slot], 1)
                pl.semaphore_wait(cred_b.at[slot], 1)

            rdma_f = pltpu.make_async_remote_copy(
                src_ref=chunk_f(sa), dst_ref=dst_f,
                send_sem=send_f.at[slot], recv_sem=recv_f.at[slot],
                device_id=(nxt,), device_id_type=pl.DeviceIdType.MESH,
            )
            rdma_b = pltpu.make_async_remote_copy(
                src_ref=chunk_b(sb), dst_ref=dst_b,
                send_sem=send_b.at[slot], recv_sem=recv_b.at[slot],
                device_id=(prv,), device_id_type=pl.DeviceIdType.MESH,
            )
            rdma_f.start()
            rdma_b.start()
            rdma_f.wait()
            rdma_b.wait()

            if rs:
                out_ref[pl.ds(ra * ch, ch), :] = (
                    out_ref[pl.ds(ra * ch, ch), :] + comm_f[slot]
                )
                out_ref[pl.ds(half + rb * ch, ch), :] = (
                    out_ref[pl.ds(half + rb * ch, ch), :] + comm_b[slot]
                )
            if step < n_steps - 2:
                pl.semaphore_signal(
                    cred_f.at[slot], inc=1,
                    device_id=(prv,), device_id_type=pl.DeviceIdType.MESH,
                )
                pl.semaphore_signal(
                    cred_b.at[slot], inc=1,
                    device_id=(nxt,), device_id_type=pl.DeviceIdType.MESH,
                )

    return pl.pallas_call(
        body,
        out_shape=jax.ShapeDtypeStruct((m, n), jnp.float32),
        in_specs=[
            pl.BlockSpec(memory_space=pltpu.VMEM),
            pl.BlockSpec(memory_space=pltpu.VMEM),
            pl.BlockSpec(memory_space=pltpu.SMEM),
            pl.BlockSpec(memory_space=pltpu.SMEM),
            pl.BlockSpec(memory_space=pltpu.SMEM),
            pl.BlockSpec(memory_space=pltpu.SMEM),
            pl.BlockSpec(memory_space=pltpu.SMEM),
        ],
        out_specs=pl.BlockSpec(memory_space=pltpu.VMEM),
        scratch_shapes=[
            pltpu.VMEM((2, ch, n), jnp.float32),
            pltpu.VMEM((2, ch, n), jnp.float32),
            pltpu.SemaphoreType.DMA((2,)),
            pltpu.SemaphoreType.DMA((2,)),
            pltpu.SemaphoreType.REGULAR((2,)),
            pltpu.SemaphoreType.DMA((2,)),
            pltpu.SemaphoreType.DMA((2,)),
            pltpu.SemaphoreType.REGULAR((2,)),
        ],
        compiler_params=pltpu.CompilerParams(
            vmem_limit_bytes=100 * 1024 * 1024,
        ),
    )(x, w_mat, scale_x, scale_w, ring_tab, succ_tab, pred_tab)


# device time: 280630 ns/iter; 2.7938x vs baseline; 1.5784x over previous
import jax
import jax.numpy as jnp
from jax import lax
from jax.experimental import pallas as pl
from jax.experimental.pallas import tpu as pltpu

N_DEV = 16

_MESH_COORDS = [(x, y, z) for z in range(4) for (x, y) in ((0, 0), (1, 0), (1, 1), (0, 1))]
_HAM = [(0, 0, 0), (0, 1, 0), (0, 1, 1), (0, 0, 1),
        (0, 0, 2), (0, 1, 2), (0, 1, 3), (0, 0, 3),
        (1, 0, 3), (1, 1, 3), (1, 1, 2), (1, 0, 2),
        (1, 0, 1), (1, 1, 1), (1, 1, 0), (1, 0, 0)]
_PI = [_MESH_COORDS.index(c) for c in _HAM]
_RING = [0] * N_DEV
_SUCC = [0] * N_DEV
_PRED = [0] * N_DEV
for _r, _m in enumerate(_PI):
    _RING[_m] = _r
    _SUCC[_m] = _PI[(_r + 1) % N_DEV]
    _PRED[_m] = _PI[(_r - 1) % N_DEV]


def kernel(x, w_mat, scale_x, scale_w):
    m, _ = x.shape
    _, n = w_mat.shape
    half = m // 2
    ch = half // N_DEV

    ring_tab = jnp.array(_RING, dtype=jnp.int32)
    succ_tab = jnp.array(_SUCC, dtype=jnp.int32)
    pred_tab = jnp.array(_PRED, dtype=jnp.int32)

    def body(x_ref, w_ref, sx_ref, sw_ref, ring_ref, succ_ref, pred_ref,
             out_ref, comm_f, comm_b, stage_f, stage_b,
             send_f, recv_f, cred_f, send_b, recv_b, cred_b):
        me = lax.axis_index("i")
        r = ring_ref[me]
        nxt = succ_ref[me]
        prv = pred_ref[me]

        s = sx_ref[0] * sw_ref[0]
        xb = x_ref[...].astype(jnp.bfloat16)
        wb = w_ref[...].astype(jnp.bfloat16)
        out_ref[...] = jnp.dot(xb, wb, preferred_element_type=jnp.float32) * s

        n_steps = 2 * (N_DEV - 1)
        for step in range(n_steps):
            slot = step % 2
            rs = step < N_DEV - 1
            if rs:
                sa = lax.rem(r - step + N_DEV, N_DEV)
                ra = lax.rem(r - step - 1 + N_DEV, N_DEV)
                sb = lax.rem(r + step, N_DEV)
                rb = lax.rem(r + step + 1, N_DEV)
            else:
                t = step - (N_DEV - 1)
                sa = lax.rem(r + 1 - t + N_DEV, N_DEV)
                ra = lax.rem(r - t + N_DEV, N_DEV)
                sb = lax.rem(r - 1 + t + N_DEV, N_DEV)
                rb = lax.rem(r + t, N_DEV)

            stage_f[slot] = out_ref[pl.ds(sa * ch, ch), :].astype(jnp.bfloat16)
            stage_b[slot] = out_ref[pl.ds(half + sb * ch, ch), :].astype(jnp.bfloat16)

            if step >= 2:
                pl.semaphore_wait(cred_f.at[slot], 1)
                pl.semaphore_wait(cred_b.at[slot], 1)

            rdma_f = pltpu.make_async_remote_copy(
                src_ref=stage_f.at[slot], dst_ref=comm_f.at[slot],
                send_sem=send_f.at[slot], recv_sem=recv_f.at[slot],
                device_id=(nxt,), device_id_type=pl.DeviceIdType.MESH,
            )
            rdma_b = pltpu.make_async_remote_copy(
                src_ref=stage_b.at[slot], dst_ref=comm_b.at[slot],
                send_sem=send_b.at[slot], recv_sem=recv_b.at[slot],
                device_id=(prv,), device_id_type=pl.DeviceIdType.MESH,
            )
            rdma_f.start()
            rdma_b.start()
            rdma_f.wait()
            rdma_b.wait()

            if rs:
                out_ref[pl.ds(ra * ch, ch), :] = (
                    out_ref[pl.ds(ra * ch, ch), :]
                    + comm_f[slot].astype(jnp.float32)
                )
                out_ref[pl.ds(half + rb * ch, ch), :] = (
                    out_ref[pl.ds(half + rb * ch, ch), :]
                    + comm_b[slot].astype(jnp.float32)
                )
            else:
                out_ref[pl.ds(ra * ch, ch), :] = comm_f[slot].astype(jnp.float32)
                out_ref[pl.ds(half + rb * ch, ch), :] = comm_b[slot].astype(jnp.float32)
            if step < n_steps - 2:
                pl.semaphore_signal(
                    cred_f.at[slot], inc=1,
                    device_id=(prv,), device_id_type=pl.DeviceIdType.MESH,
                )
                pl.semaphore_signal(
                    cred_b.at[slot], inc=1,
                    device_id=(nxt,), device_id_type=pl.DeviceIdType.MESH,
                )

    return pl.pallas_call(
        body,
        out_shape=jax.ShapeDtypeStruct((m, n), jnp.float32),
        in_specs=[
            pl.BlockSpec(memory_space=pltpu.VMEM),
            pl.BlockSpec(memory_space=pltpu.VMEM),
            pl.BlockSpec(memory_space=pltpu.SMEM),
            pl.BlockSpec(memory_space=pltpu.SMEM),
            pl.BlockSpec(memory_space=pltpu.SMEM),
            pl.BlockSpec(memory_space=pltpu.SMEM),
            pl.BlockSpec(memory_space=pltpu.SMEM),
        ],
        out_specs=pl.BlockSpec(memory_space=pltpu.VMEM),
        scratch_shapes=[
            pltpu.VMEM((2, ch, n), jnp.bfloat16),
            pltpu.VMEM((2, ch, n), jnp.bfloat16),
            pltpu.VMEM((2, ch, n), jnp.bfloat16),
            pltpu.VMEM((2, ch, n), jnp.bfloat16),
            pltpu.SemaphoreType.DMA((2,)),
            pltpu.SemaphoreType.DMA((2,)),
            pltpu.SemaphoreType.REGULAR((2,)),
            pltpu.SemaphoreType.DMA((2,)),
            pltpu.SemaphoreType.DMA((2,)),
            pltpu.SemaphoreType.REGULAR((2,)),
        ],
        compiler_params=pltpu.CompilerParams(
            vmem_limit_bytes=100 * 1024 * 1024,
        ),
    )(x, w_mat, scale_x, scale_w, ring_tab, succ_tab, pred_tab)


# device time: 220911 ns/iter; 3.5491x vs baseline; 1.2703x over previous
import jax
import jax.numpy as jnp
from jax import lax
from jax.experimental import pallas as pl
from jax.experimental.pallas import tpu as pltpu

N_DEV = 16
N_STEPS = 2 * (N_DEV - 1)

_MESH_COORDS = [(x, y, z) for z in range(4) for (x, y) in ((0, 0), (1, 0), (1, 1), (0, 1))]
_HAM = [(0, 0, 0), (0, 1, 0), (0, 1, 1), (0, 0, 1),
        (0, 0, 2), (0, 1, 2), (0, 1, 3), (0, 0, 3),
        (1, 0, 3), (1, 1, 3), (1, 1, 2), (1, 0, 2),
        (1, 0, 1), (1, 1, 1), (1, 1, 0), (1, 0, 0)]
_PI = [_MESH_COORDS.index(c) for c in _HAM]
_RING = [0] * N_DEV
_SUCC = [0] * N_DEV
_PRED = [0] * N_DEV
for _r, _m in enumerate(_PI):
    _RING[_m] = _r
    _SUCC[_m] = _PI[(_r + 1) % N_DEV]
    _PRED[_m] = _PI[(_r - 1) % N_DEV]


def kernel(x, w_mat, scale_x, scale_w):
    m, _ = x.shape
    _, n = w_mat.shape
    q = m // 4
    ch = q // N_DEV

    ring_tab = jnp.array(_RING, dtype=jnp.int32)
    succ_tab = jnp.array(_SUCC, dtype=jnp.int32)
    pred_tab = jnp.array(_PRED, dtype=jnp.int32)

    def body(x_ref, w_ref, sx_ref, sw_ref, ring_ref, succ_ref, pred_ref,
             out_ref, *scr):
        me = lax.axis_index("i")
        r = ring_ref[me]
        nxt = succ_ref[me]
        prv = pred_ref[me]

        s = sx_ref[0] * sw_ref[0]
        xb = x_ref[...].astype(jnp.bfloat16)
        wb = w_ref[...].astype(jnp.bfloat16)
        out_ref[...] = jnp.dot(xb, wb, preferred_element_type=jnp.float32) * s

        comms = scr[0:4]
        stages = scr[4:8]
        ssems = scr[8:12]
        rsems = scr[12:16]
        creds = scr[16:20]
        row0s = [0, q, 2 * q, 3 * q]
        fwds = [True, True, False, False]

        def indices(k, step):
            fwd = fwds[k]
            if step < N_DEV - 1:
                if fwd:
                    send_i = lax.rem(r - step + N_DEV, N_DEV)
                    recv_i = lax.rem(r - step - 1 + N_DEV, N_DEV)
                else:
                    send_i = lax.rem(r + step, N_DEV)
                    recv_i = lax.rem(r + step + 1, N_DEV)
            else:
                t = step - (N_DEV - 1)
                if fwd:
                    send_i = lax.rem(r + 1 - t + N_DEV, N_DEV)
                    recv_i = lax.rem(r - t + N_DEV, N_DEV)
                else:
                    send_i = lax.rem(r - 1 + t + N_DEV, N_DEV)
                    recv_i = lax.rem(r + t, N_DEV)
            return send_i, recv_i

        def start_step(k, step):
            slot = step % 2
            send_i, _ = indices(k, step)
            stages[k][slot] = out_ref[
                pl.ds(row0s[k] + send_i * ch, ch), :
            ].astype(jnp.bfloat16)
            if step >= 2:
                pl.semaphore_wait(creds[k].at[slot], 1)
            rdma = pltpu.make_async_remote_copy(
                src_ref=stages[k].at[slot], dst_ref=comms[k].at[slot],
                send_sem=ssems[k].at[slot], recv_sem=rsems[k].at[slot],
                device_id=(nxt,) if fwds[k] else (prv,),
                device_id_type=pl.DeviceIdType.MESH,
            )
            rdma.start()
            return rdma

        def finish_step(k, step, rdma):
            rdma.wait()
            slot = step % 2
            _, recv_i = indices(k, step)
            tgt = pl.ds(row0s[k] + recv_i * ch, ch)
            if step < N_DEV - 1:
                out_ref[tgt, :] = (
                    out_ref[tgt, :] + comms[k][slot].astype(jnp.float32)
                )
            else:
                out_ref[tgt, :] = comms[k][slot].astype(jnp.float32)
            if step < N_STEPS - 2:
                pl.semaphore_signal(
                    creds[k].at[slot], inc=1,
                    device_id=(prv,) if fwds[k] else (nxt,),
                    device_id_type=pl.DeviceIdType.MESH,
                )

        groups = ((0, 2), (1, 3))
        pend = [None] * 4
        for g in groups:
            for k in g:
                pend[k] = start_step(k, 0)
        for step in range(N_STEPS):
            for g in groups:
                for k in g:
                    finish_step(k, step, pend[k])
                if step + 1 < N_STEPS:
                    for k in g:
                        pend[k] = start_step(k, step + 1)

    return pl.pallas_call(
        body,
        out_shape=jax.ShapeDtypeStruct((m, n), jnp.float32),
        in_specs=[
            pl.BlockSpec(memory_space=pltpu.VMEM),
            pl.BlockSpec(memory_space=pltpu.VMEM),
            pl.BlockSpec(memory_space=pltpu.SMEM),
            pl.BlockSpec(memory_space=pltpu.SMEM),
            pl.BlockSpec(memory_space=pltpu.SMEM),
            pl.BlockSpec(memory_space=pltpu.SMEM),
            pl.BlockSpec(memory_space=pltpu.SMEM),
        ],
        out_specs=pl.BlockSpec(memory_space=pltpu.VMEM),
        scratch_shapes=(
            [pltpu.VMEM((2, ch, n), jnp.bfloat16)] * 4
            + [pltpu.VMEM((2, ch, n), jnp.bfloat16)] * 4
            + [pltpu.SemaphoreType.DMA((2,))] * 4
            + [pltpu.SemaphoreType.DMA((2,))] * 4
            + [pltpu.SemaphoreType.REGULAR((2,))] * 4
        ),
        compiler_params=pltpu.CompilerParams(
            vmem_limit_bytes=100 * 1024 * 1024,
        ),
    )(x, w_mat, scale_x, scale_w, ring_tab, succ_tab, pred_tab)


# device time: 216944 ns/iter; 3.6140x vs baseline; 1.0183x over previous
import jax
import jax.numpy as jnp
from jax import lax
from jax.experimental import pallas as pl
from jax.experimental.pallas import tpu as pltpu

N_DEV = 16
N_STEPS = 2 * (N_DEV - 1)

_MESH_COORDS = [(x, y, z) for z in range(4) for (x, y) in ((0, 0), (1, 0), (1, 1), (0, 1))]
_HAM = [(0, 0, 0), (0, 1, 0), (0, 1, 1), (0, 0, 1),
        (0, 0, 2), (0, 1, 2), (0, 1, 3), (0, 0, 3),
        (1, 0, 3), (1, 1, 3), (1, 1, 2), (1, 0, 2),
        (1, 0, 1), (1, 1, 1), (1, 1, 0), (1, 0, 0)]
_PI = [_MESH_COORDS.index(c) for c in _HAM]
_RING = [0] * N_DEV
_SUCC = [0] * N_DEV
_PRED = [0] * N_DEV
for _r, _m in enumerate(_PI):
    _RING[_m] = _r
    _SUCC[_m] = _PI[(_r + 1) % N_DEV]
    _PRED[_m] = _PI[(_r - 1) % N_DEV]


def kernel(x, w_mat, scale_x, scale_w):
    m, _ = x.shape
    _, n = w_mat.shape
    q = m // 4
    ch = q // N_DEV

    ring_tab = jnp.array(_RING, dtype=jnp.int32)
    succ_tab = jnp.array(_SUCC, dtype=jnp.int32)
    pred_tab = jnp.array(_PRED, dtype=jnp.int32)

    def body(x_ref, w_ref, sx_ref, sw_ref, ring_ref, succ_ref, pred_ref,
             out_ref, *scr):
        me = lax.axis_index("i")
        r = ring_ref[me]
        nxt = succ_ref[me]
        prv = pred_ref[me]

        s = sx_ref[0] * sw_ref[0]

        comms = scr[0:4]
        stages = scr[4:8]
        ssems = scr[8:12]
        rsems = scr[12:16]
        creds = scr[16:20]
        row0s = [0, q, 2 * q, 3 * q]
        fwds = [True, True, False, False]

        def indices(k, step):
            fwd = fwds[k]
            if step < N_DEV - 1:
                if fwd:
                    send_i = lax.rem(r - step + N_DEV, N_DEV)
                    recv_i = lax.rem(r - step - 1 + N_DEV, N_DEV)
                else:
                    send_i = lax.rem(r + step, N_DEV)
                    recv_i = lax.rem(r + step + 1, N_DEV)
            else:
                t = step - (N_DEV - 1)
                if fwd:
                    send_i = lax.rem(r + 1 - t + N_DEV, N_DEV)
                    recv_i = lax.rem(r - t + N_DEV, N_DEV)
                else:
                    send_i = lax.rem(r - 1 + t + N_DEV, N_DEV)
                    recv_i = lax.rem(r + t, N_DEV)
            return send_i, recv_i

        def start_step(k, step):
            slot = step % 2
            if step == 0:
                send_i, _ = indices(k, 0)
                stages[k][slot] = out_ref[
                    pl.ds(row0s[k] + send_i * ch, ch), :
                ].astype(jnp.bfloat16)
            if step >= 2:
                pl.semaphore_wait(creds[k].at[slot], 1)
            rdma = pltpu.make_async_remote_copy(
                src_ref=stages[k].at[slot], dst_ref=comms[k].at[slot],
                send_sem=ssems[k].at[slot], recv_sem=rsems[k].at[slot],
                device_id=(nxt,) if fwds[k] else (prv,),
                device_id_type=pl.DeviceIdType.MESH,
            )
            rdma.start()
            return rdma

        def finish_step(k, step, rdma):
            rdma.wait()
            slot = step % 2
            nslot = (step + 1) % 2
            _, recv_i = indices(k, step)
            tgt = pl.ds(row0s[k] + recv_i * ch, ch)
            arr = comms[k][slot]
            if step < N_DEV - 2:
                stages[k][nslot] = (
                    arr.astype(jnp.float32) + out_ref[tgt, :]
                ).astype(jnp.bfloat16)
            elif step == N_DEV - 2:
                acc = arr.astype(jnp.float32) + out_ref[tgt, :]
                out_ref[tgt, :] = acc
                stages[k][nslot] = acc.astype(jnp.bfloat16)
            elif step < N_STEPS - 1:
                out_ref[tgt, :] = arr.astype(jnp.float32)
                stages[k][nslot] = arr
            else:
                out_ref[tgt, :] = arr.astype(jnp.float32)
            if step < N_STEPS - 2:
                pl.semaphore_signal(
                    creds[k].at[slot], inc=1,
                    device_id=(prv,) if fwds[k] else (nxt,),
                    device_id_type=pl.DeviceIdType.MESH,
                )

        groups = ((0, 2), (1, 3))

        wb = w_ref[...].astype(jnp.bfloat16)
        for k in range(4):
            send_i, _ = indices(k, 0)
            rows = pl.ds(row0s[k] + send_i * ch, ch)
            out_ref[rows, :] = (
                jnp.dot(x_ref[rows, :].astype(jnp.bfloat16), wb,
                        preferred_element_type=jnp.float32) * s
            )
        pend = [None] * 4
        for g in groups:
            for k in g:
                pend[k] = start_step(k, 0)
        xb = x_ref[...].astype(jnp.bfloat16)
        out_ref[...] = jnp.dot(xb, wb, preferred_element_type=jnp.float32) * s

        for step in range(N_STEPS):
            for g in groups:
                for k in g:
                    finish_step(k, step, pend[k])
                if step + 1 < N_STEPS:
                    for k in g:
                        pend[k] = start_step(k, step + 1)

    return pl.pallas_call(
        body,
        out_shape=jax.ShapeDtypeStruct((m, n), jnp.float32),
        in_specs=[
            pl.BlockSpec(memory_space=pltpu.VMEM),
            pl.BlockSpec(memory_space=pltpu.VMEM),
            pl.BlockSpec(memory_space=pltpu.SMEM),
            pl.BlockSpec(memory_space=pltpu.SMEM),
            pl.BlockSpec(memory_space=pltpu.SMEM),
            pl.BlockSpec(memory_space=pltpu.SMEM),
            pl.BlockSpec(memory_space=pltpu.SMEM),
        ],
        out_specs=pl.BlockSpec(memory_space=pltpu.VMEM),
        scratch_shapes=(
            [pltpu.VMEM((2, ch, n), jnp.bfloat16)] * 4
            + [pltpu.VMEM((2, ch, n), jnp.bfloat16)] * 4
            + [pltpu.SemaphoreType.DMA((2,))] * 4
            + [pltpu.SemaphoreType.DMA((2,))] * 4
            + [pltpu.SemaphoreType.REGULAR((2,))] * 4
        ),
        compiler_params=pltpu.CompilerParams(
            vmem_limit_bytes=100 * 1024 * 1024,
        ),
    )(x, w_mat, scale_x, scale_w, ring_tab, succ_tab, pred_tab)
